# Initial kernel scaffold; baseline (speedup 1.0000x reference)
import jax
import jax.numpy as jnp
from jax.experimental import pallas as pl


def kernel(x, Wg, bg, W1, b1, W2, b2, expert_bias):
    raise NotImplementedError("write your pallas kernel here")


# speedup vs baseline: 1.5157x; 1.5157x over previous
import functools

import jax
import jax.numpy as jnp
from jax import lax
from jax.experimental import pallas as pl
from jax.experimental.pallas import tpu as pltpu
from jax.experimental.pallas import tpu_sc as plsc

BT = 256
BF = 512


def _gate_body(x_ref, wg_ref, bias_ref, out_ref):
    logits = lax.dot_general(x_ref[...], wg_ref[...],
                             (((1,), (1,)), ((), ())),
                             preferred_element_type=jnp.float32)
    logits = logits + bias_ref[...]
    probs = jax.nn.softmax(logits, axis=-1)
    t, e = probs.shape
    m = jnp.max(probs, axis=-1, keepdims=True)
    ii = lax.broadcasted_iota(jnp.int32, (t, e), 1)
    cand = jnp.where(probs >= m, ii, e)
    out_ref[...] = jnp.min(cand, axis=-1).astype(jnp.int32)


def _gating(flat, wg, bias):
    t, d = flat.shape
    e = wg.shape[0]
    return pl.pallas_call(
        _gate_body,
        out_shape=jax.ShapeDtypeStruct((t,), jnp.int32),
    )(flat, wg, bias.reshape(1, e))


def _make_row_gather(n_rows, n_idx, d):
    nw = 32
    ch = 64
    n_per = n_idx // nw
    assert n_idx % (nw * ch) == 0
    mesh = plsc.VectorSubcoreMesh(core_axis_name="c", subcore_axis_name="s")

    @functools.partial(
        pl.kernel, mesh=mesh,
        out_type=jax.ShapeDtypeStruct((n_idx, d), jnp.float32),
        scratch_types=[
            pltpu.VMEM((ch,), jnp.int32),
            pltpu.VMEM((ch, d), jnp.float32),
            pltpu.SemaphoreType.DMA,
        ],
    )
    def gather(table_hbm, idx_hbm, out_hbm, idx_v, rows_v, sem):
        wid = lax.axis_index("s") * 2 + lax.axis_index("c")
        base = wid * n_per

        def body(c, carry):
            off = pl.multiple_of(base + c * ch, ch)
            pltpu.sync_copy(idx_hbm.at[pl.ds(off, ch)], idx_v)
            pltpu.async_copy(table_hbm.at[idx_v], rows_v, sem).wait()
            pltpu.sync_copy(rows_v, out_hbm.at[pl.ds(off, ch)])
            return carry

        lax.fori_loop(0, n_per // ch, body, 0)

    return gather


def _ffn_body(be_ref, nu_ref, xs_ref, w1_ref, b1_ref, w2_ref, b2_ref,
              out_ref, acc_ref, *, n_ff_blocks):
    i = pl.program_id(0)
    k = pl.program_id(1)

    @pl.when(i < nu_ref[0])
    def _():
        h = lax.dot_general(xs_ref[...], w1_ref[0],
                            (((1,), (1,)), ((), ())),
                            preferred_element_type=jnp.float32)
        h = jnp.maximum(h + b1_ref[0, 0], 0.0)
        part = lax.dot_general(h, w2_ref[0],
                               (((1,), (1,)), ((), ())),
                               preferred_element_type=jnp.float32)

        @pl.when(k == 0)
        def _():
            acc_ref[...] = part

        @pl.when(k > 0)
        def _():
            acc_ref[...] += part

        @pl.when(k == n_ff_blocks - 1)
        def _():
            out_ref[...] = acc_ref[...] + b2_ref[0]


def _ffn(xs, w1, b1, w2, b2, block_expert, n_used):
    p, d = xs.shape
    e, d_ff, _ = w1.shape
    nblk = p // BT
    kk = d_ff // BF
    b1r = b1.reshape(e, kk, 1, BF)
    b2r = b2.reshape(e, 1, d)
    grid_spec = pltpu.PrefetchScalarGridSpec(
        num_scalar_prefetch=2,
        grid=(nblk, kk),
        in_specs=[
            pl.BlockSpec((BT, d), lambda i, k, be, nu: (i, 0)),
            pl.BlockSpec((1, BF, d), lambda i, k, be, nu: (be[i], k, 0)),
            pl.BlockSpec((1, 1, 1, BF), lambda i, k, be, nu: (be[i], k, 0, 0)),
            pl.BlockSpec((1, d, BF), lambda i, k, be, nu: (be[i], 0, k)),
            pl.BlockSpec((1, 1, d), lambda i, k, be, nu: (be[i], 0, 0)),
        ],
        out_specs=pl.BlockSpec((BT, d), lambda i, k, be, nu: (i, 0)),
        scratch_shapes=[pltpu.VMEM((BT, d), jnp.float32)],
    )
    return pl.pallas_call(
        functools.partial(_ffn_body, n_ff_blocks=kk),
        grid_spec=grid_spec,
        out_shape=jax.ShapeDtypeStruct((p, d), jnp.float32),
        compiler_params=pltpu.CompilerParams(
            dimension_semantics=("arbitrary", "arbitrary")),
    )(block_expert, n_used, xs, w1, b1r, w2, b2r)


def kernel(x, Wg, bg, W1, b1, W2, b2, expert_bias):
    b, s, d = x.shape
    e, d_ff, _ = W1.shape
    t = b * s
    p = t + e * BT
    flat = x.reshape(t, d)

    top_expert = _gating(flat, Wg, bg + expert_bias)

    order = jnp.argsort(top_expert, stable=True)
    te_sorted = top_expert[order]
    counts = jnp.bincount(top_expert, length=e)
    cap = ((counts + BT - 1) // BT) * BT
    cap_cum = jnp.cumsum(cap)
    cap_start = cap_cum - cap
    cnt_start = jnp.cumsum(counts) - counts
    r = jnp.arange(t, dtype=jnp.int32)
    dst = (cap_start[te_sorted] - cnt_start[te_sorted]).astype(jnp.int32) + r
    src = jnp.zeros((p,), jnp.int32).at[dst].set(order.astype(jnp.int32))
    unsort = jnp.zeros((t,), jnp.int32).at[order].set(dst)
    nblk = p // BT
    blk_off = jnp.arange(nblk, dtype=jnp.int32) * BT
    total_cap = cap_cum[-1]
    block_expert = jnp.searchsorted(
        cap_cum, jnp.minimum(blk_off, total_cap - 1), side="right"
    ).astype(jnp.int32)
    n_used = (total_cap // BT).astype(jnp.int32).reshape(1)

    xs = _make_row_gather(t, p, d)(flat, src)
    ys = _ffn(xs, W1, b1, W2, b2, block_expert, n_used)
    out = _make_row_gather(p, t, d)(ys, unsort)
    return out.reshape(b, s, d)


# speedup vs baseline: 2.8343x; 1.8700x over previous
import functools

import jax
import jax.numpy as jnp
from jax import lax
from jax.experimental import pallas as pl
from jax.experimental.pallas import tpu as pltpu
from jax.experimental.pallas import tpu_sc as plsc

BT = 256


def _gate_body(x_ref, wg_ref, bias_ref, out_ref):
    logits = lax.dot_general(x_ref[...], wg_ref[...],
                             (((1,), (1,)), ((), ())),
                             preferred_element_type=jnp.float32)
    logits = logits + bias_ref[...]
    probs = jax.nn.softmax(logits, axis=-1)
    t, e = probs.shape
    m = jnp.max(probs, axis=-1, keepdims=True)
    ii = lax.broadcasted_iota(jnp.int32, (t, e), 1)
    cand = jnp.where(probs >= m, ii, e)
    out_ref[...] = jnp.min(cand, axis=-1).astype(jnp.int32)


def _gating(flat, wg, bias):
    t, d = flat.shape
    e = wg.shape[0]
    return pl.pallas_call(
        _gate_body,
        out_shape=jax.ShapeDtypeStruct((t,), jnp.int32),
    )(flat, wg, bias.reshape(1, e))


def _make_row_scatter(n_src, n_out, d):
    nw = 32
    ch = 64
    n_per = n_src // nw
    assert n_src % (nw * ch) == 0
    mesh = plsc.VectorSubcoreMesh(core_axis_name="c", subcore_axis_name="s")

    @functools.partial(
        pl.kernel, mesh=mesh,
        out_type=jax.ShapeDtypeStruct((n_out, d), jnp.float32),
        scratch_types=[
            pltpu.VMEM((ch,), jnp.int32),
            pltpu.VMEM((ch, d), jnp.float32),
            pltpu.SemaphoreType.DMA,
        ],
    )
    def scatter(table_hbm, idx_hbm, out_hbm, idx_v, rows_v, sem):
        wid = lax.axis_index("s") * 2 + lax.axis_index("c")
        base = wid * n_per

        def body(c, carry):
            off = pl.multiple_of(base + c * ch, ch)
            pltpu.sync_copy(idx_hbm.at[pl.ds(off, ch)], idx_v)
            pltpu.sync_copy(table_hbm.at[pl.ds(off, ch)], rows_v)
            pltpu.async_copy(rows_v, out_hbm.at[idx_v], sem).wait()
            return carry

        lax.fori_loop(0, n_per // ch, body, 0)

    return scatter


def _make_row_gather(n_rows, n_idx, d):
    nw = 32
    ch = 64
    n_per = n_idx // nw
    assert n_idx % (nw * ch) == 0
    mesh = plsc.VectorSubcoreMesh(core_axis_name="c", subcore_axis_name="s")

    @functools.partial(
        pl.kernel, mesh=mesh,
        out_type=jax.ShapeDtypeStruct((n_idx, d), jnp.float32),
        scratch_types=[
            pltpu.VMEM((ch,), jnp.int32),
            pltpu.VMEM((ch, d), jnp.float32),
            pltpu.SemaphoreType.DMA,
        ],
    )
    def gather(table_hbm, idx_hbm, out_hbm, idx_v, rows_v, sem):
        wid = lax.axis_index("s") * 2 + lax.axis_index("c")
        base = wid * n_per

        def body(c, carry):
            off = pl.multiple_of(base + c * ch, ch)
            pltpu.sync_copy(idx_hbm.at[pl.ds(off, ch)], idx_v)
            pltpu.async_copy(table_hbm.at[idx_v], rows_v, sem).wait()
            pltpu.sync_copy(rows_v, out_hbm.at[pl.ds(off, ch)])
            return carry

        lax.fori_loop(0, n_per // ch, body, 0)

    return gather


KS = 2


def _ffn_body(be_ref, nu_ref, xs_ref, w1_ref, b1_ref, w2_ref, b2_ref, out_ref):
    k = pl.program_id(0)
    i = pl.program_id(1)

    @pl.when(i < nu_ref[0])
    def _():
        h = lax.dot_general(xs_ref[...], w1_ref[0],
                            (((1,), (1,)), ((), ())),
                            preferred_element_type=jnp.float32)
        h = jnp.maximum(h + b1_ref[0, 0], 0.0)
        part = lax.dot_general(h, w2_ref[0],
                               (((1,), (1,)), ((), ())),
                               preferred_element_type=jnp.float32)
        scale = jnp.where(k == 0, 1.0, 0.0)
        out_ref[0] = part + b2_ref[0] * scale


def _ffn(xs, w1, b1, w2, b2, block_expert, n_used):
    p, d = xs.shape
    e, d_ff, _ = w1.shape
    nblk = p // BT
    dff2 = d_ff // KS
    b1r = b1.reshape(e, KS, 1, dff2)
    b2r = b2.reshape(e, 1, d)
    grid_spec = pltpu.PrefetchScalarGridSpec(
        num_scalar_prefetch=2,
        grid=(KS, nblk),
        in_specs=[
            pl.BlockSpec((BT, d), lambda k, i, be, nu: (i, 0)),
            pl.BlockSpec((1, dff2, d), lambda k, i, be, nu: (be[i], k, 0)),
            pl.BlockSpec((1, 1, 1, dff2), lambda k, i, be, nu: (be[i], k, 0, 0)),
            pl.BlockSpec((1, d, dff2), lambda k, i, be, nu: (be[i], 0, k)),
            pl.BlockSpec((1, 1, d), lambda k, i, be, nu: (be[i], 0, 0)),
        ],
        out_specs=pl.BlockSpec((1, BT, d), lambda k, i, be, nu: (k, i, 0)),
    )
    return pl.pallas_call(
        _ffn_body,
        grid_spec=grid_spec,
        out_shape=jax.ShapeDtypeStruct((KS, p, d), jnp.float32),
        compiler_params=pltpu.CompilerParams(
            dimension_semantics=("arbitrary", "arbitrary")),
    )(block_expert, n_used, xs, w1, b1r, w2, b2r)


def kernel(x, Wg, bg, W1, b1, W2, b2, expert_bias):
    b, s, d = x.shape
    e, d_ff, _ = W1.shape
    t = b * s
    p = t + e * BT
    flat = x.reshape(t, d)

    top_expert = _gating(flat, Wg, bg + expert_bias)

    onehot = (top_expert[:, None] == jnp.arange(e, dtype=jnp.int32)[None, :])
    pc = jnp.cumsum(onehot.astype(jnp.int32), axis=0)
    counts = pc[-1]
    rank = jnp.take_along_axis(pc, top_expert[:, None], axis=1)[:, 0] - 1
    cap = ((counts + BT - 1) // BT) * BT
    cap_cum = jnp.cumsum(cap)
    dst = ((cap_cum - cap)[top_expert] + rank).astype(jnp.int32)
    nblk = p // BT
    blk_off = jnp.arange(nblk, dtype=jnp.int32) * BT
    total_cap = cap_cum[-1]
    block_expert = jnp.searchsorted(
        cap_cum, jnp.minimum(blk_off, total_cap - 1), side="right"
    ).astype(jnp.int32)
    n_used = (total_cap // BT).astype(jnp.int32).reshape(1)

    xs = _make_row_scatter(t, p, d)(flat, dst)
    ys = _ffn(xs, W1, b1, W2, b2, block_expert, n_used)
    out = _make_row_gather(p, t, d)(ys[0] + ys[1], dst)
    return out.reshape(b, s, d)
